# initial kernel scaffold (unmeasured)
import jax
import jax.numpy as jnp
from jax import lax
from jax.experimental import pallas as pl
from jax.experimental.pallas import tpu as pltpu


def kernel(
    x,
):
    def body(*refs):
        pass

    out_shape = jax.ShapeDtypeStruct(..., jnp.float32)
    return pl.pallas_call(body, out_shape=out_shape)(...)


# baseline (device time: 99715 ns/iter reference)
import jax
import jax.numpy as jnp
from jax import lax
from jax.experimental import pallas as pl
from jax.experimental.pallas import tpu as pltpu

N_DEV = 8


def kernel(x):
    m_per, n = x.shape

    def body(x_ref, out_ref, comm_ref, send_sems, recv_sems):
        my_pos = lax.axis_index("i")
        left = (my_pos - 1) % N_DEV
        right = (my_pos + 1) % N_DEV

        barrier_sem = pltpu.get_barrier_semaphore()
        for nbr in [left, right]:
            pl.semaphore_signal(
                barrier_sem, inc=1,
                device_id=(nbr,), device_id_type=pl.DeviceIdType.MESH,
            )
        pl.semaphore_wait(barrier_sem, 2)

        out_ref[pl.ds(my_pos * m_per, m_per), :] = x_ref[:, :]
        comm_ref[0, :, :] = x_ref[:, :]

        for h in range(N_DEV - 1):
            send_slot = h % 2
            recv_slot = (h + 1) % 2
            rdma = pltpu.make_async_remote_copy(
                src_ref=comm_ref.at[send_slot],
                dst_ref=comm_ref.at[recv_slot],
                send_sem=send_sems.at[send_slot],
                recv_sem=recv_sems.at[recv_slot],
                device_id=(right,),
                device_id_type=pl.DeviceIdType.MESH,
            )
            rdma.start()
            rdma.wait()

            origin = (my_pos - h - 1) % N_DEV
            out_ref[pl.ds(origin * m_per, m_per), :] = comm_ref[recv_slot, :, :]

    return pl.pallas_call(
        body,
        out_shape=jax.ShapeDtypeStruct((N_DEV * m_per, n), x.dtype),
        in_specs=[pl.BlockSpec(memory_space=pltpu.VMEM)],
        out_specs=pl.BlockSpec(memory_space=pltpu.VMEM),
        scratch_shapes=[
            pltpu.VMEM((2, m_per, n), x.dtype),
            pltpu.SemaphoreType.DMA((2,)),
            pltpu.SemaphoreType.DMA((2,)),
        ],
        compiler_params=pltpu.CompilerParams(collective_id=0),
    )(x)


# device time: 39572 ns/iter; 2.5198x vs baseline; 2.5198x over previous
import jax
import jax.numpy as jnp
from jax import lax
from jax.experimental import pallas as pl
from jax.experimental.pallas import tpu as pltpu

N_DEV = 8
TH_OFF = (0, 176, 352)
TH_SZ = (176, 176, 160)


def _logical(x, y, z):
    return 4 * z + (jnp.bitwise_xor(x, y) + 2 * y)


def _coords(i):
    z = i // 4
    r = i % 4
    y = r // 2
    x = jnp.bitwise_xor(r, y) % 2
    return x, y, z


def kernel(x):
    m_per, n = x.shape

    def body(x_ref, out_ref, send_sems, recv_sems):
        my_pos = lax.axis_index("i")
        mx, my, mz = _coords(my_pos)

        nb = [
            _logical(1 - mx, my, mz),
            _logical(mx, 1 - my, mz),
            _logical(mx, my, 1 - mz),
        ]
        c2 = [
            _logical(1 - mx, 1 - my, mz),
            _logical(mx, 1 - my, 1 - mz),
            _logical(1 - mx, my, 1 - mz),
        ]
        anti = _logical(1 - mx, 1 - my, 1 - mz)

        barrier_sem = pltpu.get_barrier_semaphore()
        for a in range(3):
            pl.semaphore_signal(
                barrier_sem, inc=1,
                device_id=(nb[a],), device_id_type=pl.DeviceIdType.MESH,
            )
        pl.semaphore_wait(barrier_sem, 3)

        out_ref[pl.ds(my_pos * m_per, m_per), :] = x_ref[:, :]

        sends = []

        for a in range(3):
            rdma = pltpu.make_async_remote_copy(
                src_ref=x_ref,
                dst_ref=out_ref.at[pl.ds(my_pos * m_per, m_per), :],
                send_sem=send_sems.at[0, a],
                recv_sem=recv_sems.at[0, a],
                device_id=(nb[a],),
                device_id_type=pl.DeviceIdType.MESH,
            )
            rdma.start()
            sends.append(rdma)

        for a in range(3):
            recv = pltpu.make_async_remote_copy(
                src_ref=x_ref,
                dst_ref=out_ref.at[pl.ds(nb[a] * m_per, m_per), :],
                send_sem=send_sems.at[0, a],
                recv_sem=recv_sems.at[0, a],
                device_id=(nb[a],),
                device_id_type=pl.DeviceIdType.MESH,
            )
            recv.wait_recv()

        for a in range(3):
            w = nb[(a + 1) % 3]
            rdma = pltpu.make_async_remote_copy(
                src_ref=out_ref.at[pl.ds(w * m_per, m_per), :],
                dst_ref=out_ref.at[pl.ds(w * m_per, m_per), :],
                send_sem=send_sems.at[1, a],
                recv_sem=recv_sems.at[1, a],
                device_id=(nb[a],),
                device_id_type=pl.DeviceIdType.MESH,
            )
            rdma.start()
            sends.append(rdma)

        for a in range(3):
            recv = pltpu.make_async_remote_copy(
                src_ref=out_ref.at[pl.ds(c2[a] * m_per, m_per), :],
                dst_ref=out_ref.at[pl.ds(c2[a] * m_per, m_per), :],
                send_sem=send_sems.at[1, a],
                recv_sem=recv_sems.at[1, a],
                device_id=(nb[a],),
                device_id_type=pl.DeviceIdType.MESH,
            )
            recv.wait_recv()

        for a in range(3):
            ch = c2[(a + 1) % 3]
            rdma = pltpu.make_async_remote_copy(
                src_ref=out_ref.at[pl.ds(ch * m_per + TH_OFF[a], TH_SZ[a]), :],
                dst_ref=out_ref.at[pl.ds(ch * m_per + TH_OFF[a], TH_SZ[a]), :],
                send_sem=send_sems.at[2, a],
                recv_sem=recv_sems.at[2, a],
                device_id=(nb[a],),
                device_id_type=pl.DeviceIdType.MESH,
            )
            rdma.start()
            sends.append(rdma)

        for a in range(3):
            recv = pltpu.make_async_remote_copy(
                src_ref=out_ref.at[pl.ds(anti * m_per + TH_OFF[a], TH_SZ[a]), :],
                dst_ref=out_ref.at[pl.ds(anti * m_per + TH_OFF[a], TH_SZ[a]), :],
                send_sem=send_sems.at[2, a],
                recv_sem=recv_sems.at[2, a],
                device_id=(nb[a],),
                device_id_type=pl.DeviceIdType.MESH,
            )
            recv.wait_recv()

        for rdma in sends:
            rdma.wait_send()

    return pl.pallas_call(
        body,
        out_shape=jax.ShapeDtypeStruct((N_DEV * m_per, n), x.dtype),
        in_specs=[pl.BlockSpec(memory_space=pltpu.VMEM)],
        out_specs=pl.BlockSpec(memory_space=pltpu.VMEM),
        scratch_shapes=[
            pltpu.SemaphoreType.DMA((3, 3)),
            pltpu.SemaphoreType.DMA((3, 3)),
        ],
        compiler_params=pltpu.CompilerParams(collective_id=0),
    )(x)


# device time: 38274 ns/iter; 2.6053x vs baseline; 1.0339x over previous
import jax
import jax.numpy as jnp
from jax import lax
from jax.experimental import pallas as pl
from jax.experimental.pallas import tpu as pltpu

N_DEV = 8
TH_OFF = (0, 176, 352)
TH_SZ = (176, 176, 160)


def _logical(x, y, z):
    return 4 * z + (jnp.bitwise_xor(x, y) + 2 * y)


def _coords(i):
    z = i // 4
    r = i % 4
    y = r // 2
    x = jnp.bitwise_xor(r, y) % 2
    return x, y, z


def kernel(x):
    m_per, n = x.shape

    def body(x_ref, out_ref, send_sems, recv_sems):
        my_pos = lax.axis_index("i")
        mx, my, mz = _coords(my_pos)

        nb = [
            _logical(1 - mx, my, mz),
            _logical(mx, 1 - my, mz),
            _logical(mx, my, 1 - mz),
        ]
        c2 = [
            _logical(1 - mx, 1 - my, mz),
            _logical(mx, 1 - my, 1 - mz),
            _logical(1 - mx, my, 1 - mz),
        ]
        anti = _logical(1 - mx, 1 - my, 1 - mz)

        def copy(chunk, sub, phase, axis, target):
            sl = pl.ds(chunk * m_per + TH_OFF[sub], TH_SZ[sub])
            return pltpu.make_async_remote_copy(
                src_ref=out_ref.at[sl, :],
                dst_ref=out_ref.at[sl, :],
                send_sem=send_sems.at[phase, axis, sub],
                recv_sem=recv_sems.at[phase, axis, sub],
                device_id=(target,),
                device_id_type=pl.DeviceIdType.MESH,
            )

        barrier_sem = pltpu.get_barrier_semaphore()
        for a in range(3):
            pl.semaphore_signal(
                barrier_sem, inc=1,
                device_id=(nb[a],), device_id_type=pl.DeviceIdType.MESH,
            )
        pl.semaphore_wait(barrier_sem, 3)

        out_ref[pl.ds(my_pos * m_per, m_per), :] = x_ref[:, :]

        sends = []

        for s in range(3):
            for a in range(3):
                rdma = pltpu.make_async_remote_copy(
                    src_ref=x_ref.at[pl.ds(TH_OFF[s], TH_SZ[s]), :],
                    dst_ref=out_ref.at[
                        pl.ds(my_pos * m_per + TH_OFF[s], TH_SZ[s]), :
                    ],
                    send_sem=send_sems.at[0, a, s],
                    recv_sem=recv_sems.at[0, a, s],
                    device_id=(nb[a],),
                    device_id_type=pl.DeviceIdType.MESH,
                )
                rdma.start()
                sends.append(rdma)

        for s in range(3):
            for a in range(3):
                b = (a + 1) % 3
                copy(nb[b], s, 0, b, nb[b]).wait_recv()
                rdma = copy(nb[b], s, 1, a, nb[a])
                rdma.start()
                sends.append(rdma)

        for a in range(3):
            b = (a + 1) % 3
            copy(c2[b], a, 1, b, nb[b]).wait_recv()
            rdma = copy(c2[b], a, 2, a, nb[a])
            rdma.start()
            sends.append(rdma)

        for s in range(3):
            for b in range(3):
                if s == (b - 1) % 3:
                    continue
                copy(c2[b], s, 1, b, nb[b]).wait_recv()

        for a in range(3):
            copy(anti, a, 2, a, nb[a]).wait_recv()

        for rdma in sends:
            rdma.wait_send()

    return pl.pallas_call(
        body,
        out_shape=jax.ShapeDtypeStruct((N_DEV * m_per, n), x.dtype),
        in_specs=[pl.BlockSpec(memory_space=pltpu.VMEM)],
        out_specs=pl.BlockSpec(memory_space=pltpu.VMEM),
        scratch_shapes=[
            pltpu.SemaphoreType.DMA((3, 3, 3)),
            pltpu.SemaphoreType.DMA((3, 3, 3)),
        ],
        compiler_params=pltpu.CompilerParams(collective_id=0),
    )(x)


# device time: 36480 ns/iter; 2.7334x vs baseline; 1.0492x over previous
import jax
import jax.numpy as jnp
from jax import lax
from jax.experimental import pallas as pl
from jax.experimental.pallas import tpu as pltpu

N_DEV = 8
TH_OFF = (0, 176, 352)
TH_SZ = (176, 176, 160)


def _logical(x, y, z):
    return 4 * z + (jnp.bitwise_xor(x, y) + 2 * y)


def _coords(i):
    z = i // 4
    r = i % 4
    y = r // 2
    x = jnp.bitwise_xor(r, y) % 2
    return x, y, z


def kernel(x):
    m_per, n = x.shape

    def body(x_ref, out_ref, send_sems, recv_sems):
        my_pos = lax.axis_index("i")
        mx, my, mz = _coords(my_pos)

        nb = [
            _logical(1 - mx, my, mz),
            _logical(mx, 1 - my, mz),
            _logical(mx, my, 1 - mz),
        ]
        c2 = [
            _logical(1 - mx, 1 - my, mz),
            _logical(mx, 1 - my, 1 - mz),
            _logical(1 - mx, my, 1 - mz),
        ]
        anti = _logical(1 - mx, 1 - my, 1 - mz)

        def copy(chunk, sub, phase, axis, target):
            sl = pl.ds(chunk * m_per + TH_OFF[sub], TH_SZ[sub])
            return pltpu.make_async_remote_copy(
                src_ref=out_ref.at[sl, :],
                dst_ref=out_ref.at[sl, :],
                send_sem=send_sems.at[phase, axis, sub],
                recv_sem=recv_sems.at[phase, axis, sub],
                device_id=(target,),
                device_id_type=pl.DeviceIdType.MESH,
            )

        barrier_sem = pltpu.get_barrier_semaphore()
        for a in range(3):
            pl.semaphore_signal(
                barrier_sem, inc=1,
                device_id=(nb[a],), device_id_type=pl.DeviceIdType.MESH,
            )
        pl.semaphore_wait(barrier_sem, 3)

        sends = []

        for k in range(3):
            for a in range(3):
                s = (a + k) % 3
                rdma = pltpu.make_async_remote_copy(
                    src_ref=x_ref.at[pl.ds(TH_OFF[s], TH_SZ[s]), :],
                    dst_ref=out_ref.at[
                        pl.ds(my_pos * m_per + TH_OFF[s], TH_SZ[s]), :
                    ],
                    send_sem=send_sems.at[0, a, s],
                    recv_sem=recv_sems.at[0, a, s],
                    device_id=(nb[a],),
                    device_id_type=pl.DeviceIdType.MESH,
                )
                rdma.start()
                sends.append(rdma)

        out_ref[pl.ds(my_pos * m_per, m_per), :] = x_ref[:, :]

        for k in range(3):
            for a in range(3):
                b = (a + 1) % 3
                s = (b + k) % 3
                copy(nb[b], s, 0, b, nb[b]).wait_recv()
                rdma = copy(nb[b], s, 1, a, nb[a])
                rdma.start()
                sends.append(rdma)

        for a in range(3):
            b = (a + 1) % 3
            s3 = (a + 2) % 3
            copy(c2[b], s3, 1, b, nb[b]).wait_recv()
            rdma = copy(c2[b], s3, 2, a, nb[a])
            rdma.start()
            sends.append(rdma)

        for k in range(1, 3):
            for b in range(3):
                s = (b + 1 + k) % 3
                copy(c2[b], s, 1, b, nb[b]).wait_recv()

        for a in range(3):
            copy(anti, (a + 2) % 3, 2, a, nb[a]).wait_recv()

        for rdma in sends:
            rdma.wait_send()

    return pl.pallas_call(
        body,
        out_shape=jax.ShapeDtypeStruct((N_DEV * m_per, n), x.dtype),
        in_specs=[pl.BlockSpec(memory_space=pltpu.VMEM)],
        out_specs=pl.BlockSpec(memory_space=pltpu.VMEM),
        scratch_shapes=[
            pltpu.SemaphoreType.DMA((3, 3, 3)),
            pltpu.SemaphoreType.DMA((3, 3, 3)),
        ],
        compiler_params=pltpu.CompilerParams(collective_id=0),
    )(x)
